# baseline (speedup 1.0000x reference)
import jax
import jax.numpy as jnp
from jax.experimental import pallas as pl
from jax.experimental.pallas import tpu as pltpu

_TOPN = 12


def _chain_topk_kernel(x_ref, v0_ref, out_ref, acc):
    t = pl.program_id(1)

    @pl.when(t == 0)
    def _init():
        acc[...] = x_ref[0, 0]

    @pl.when(t > 0)
    def _step():
        a = x_ref[0, 0].astype(jnp.bfloat16)
        b = acc[...].astype(jnp.bfloat16)
        acc[...] = jax.lax.dot_general(
            a, b, (((1,), (0,)), ((), ())),
            preferred_element_type=jnp.float32)

    @pl.when(t == 10)
    def _finish():
        v = v0_ref[0].astype(jnp.bfloat16)
        m = acc[...].astype(jnp.bfloat16)
        row = jax.lax.dot_general(
            v, m, (((1,), (0,)), ((), ())),
            preferred_element_type=jnp.float32)

        def topk_indices(seg, base):
            idxs = jax.lax.broadcasted_iota(jnp.int32, seg.shape, 1)
            picks = []
            cur = seg
            for _ in range(_TOPN):
                mx = jnp.max(cur, axis=1, keepdims=True)
                ind = jnp.min(
                    jnp.where(cur == mx, idxs, jnp.int32(2**30)),
                    axis=1, keepdims=True)
                picks.append(ind + base)
                cur = jnp.where(idxs == ind, -jnp.inf, cur)
            return picks

        p0 = topk_indices(row[:, 1:197], 1)
        p1 = topk_indices(row[:, 198:394], 198)
        out_ref[0] = jnp.concatenate(p0 + p1, axis=1).astype(jnp.int32)


def kernel(x, topn):
    length, bsz, heads, n, _ = x.shape
    nb = bsz * heads
    xr = x.reshape(length, nb, n, n)
    v0 = xr[length - 1, :, 0, :].reshape(nb, 1, n)

    out = pl.pallas_call(
        _chain_topk_kernel,
        grid=(nb, length - 1),
        in_specs=[
            pl.BlockSpec((1, 1, n, n), lambda b, t: (t, b, 0, 0)),
            pl.BlockSpec((1, 1, n), lambda b, t: (b, 0, 0)),
        ],
        out_specs=pl.BlockSpec((1, 1, 2 * _TOPN), lambda b, t: (b, 0, 0)),
        out_shape=jax.ShapeDtypeStruct((nb, 1, 2 * _TOPN), jnp.int32),
        scratch_shapes=[pltpu.VMEM((n, n), jnp.float32)],
        compiler_params=pltpu.CompilerParams(
            dimension_semantics=("parallel", "arbitrary")),
    )(xr, v0)

    out = out.reshape(bsz, heads, 2 * _TOPN)
    shift = jnp.asarray(topn, jnp.int32) - _TOPN
    out0 = out[:, :, :_TOPN].reshape(bsz, heads * _TOPN)
    out1 = out[:, :, _TOPN:].reshape(bsz, heads * _TOPN)
    return jnp.concatenate([out0 + shift, out1 + shift], axis=1)


# speedup vs baseline: 1.3572x; 1.3572x over previous
import jax
import jax.numpy as jnp
from jax.experimental import pallas as pl
from jax.experimental.pallas import tpu as pltpu

_TOPN = 12


def _chain_topk_kernel(x_ref, v0_ref, out_ref):
    acc = x_ref[0, 0]
    for t in range(1, 11):
        a = x_ref[t, 0].astype(jnp.bfloat16)
        acc = jax.lax.dot_general(
            a, acc.astype(jnp.bfloat16), (((1,), (0,)), ((), ())),
            preferred_element_type=jnp.float32)

    v = v0_ref[0].astype(jnp.bfloat16)
    row = jax.lax.dot_general(
        v, acc.astype(jnp.bfloat16), (((1,), (0,)), ((), ())),
        preferred_element_type=jnp.float32)

    def topk_indices(seg, base):
        idxs = jax.lax.broadcasted_iota(jnp.int32, seg.shape, 1)
        picks = []
        cur = seg
        for _ in range(_TOPN):
            mx = jnp.max(cur, axis=1, keepdims=True)
            ind = jnp.min(
                jnp.where(cur == mx, idxs, jnp.int32(2**30)),
                axis=1, keepdims=True)
            picks.append(ind + base)
            cur = jnp.where(idxs == ind, -jnp.inf, cur)
        return picks

    p0 = topk_indices(row[:, 1:197], 1)
    p1 = topk_indices(row[:, 198:394], 198)
    out_ref[0] = jnp.concatenate(p0 + p1, axis=1).astype(jnp.int32)


def kernel(x, topn):
    length, bsz, heads, n, _ = x.shape
    nb = bsz * heads
    xr = x.reshape(length, nb, n, n)
    v0 = xr[length - 1, :, 0, :].reshape(nb, 1, n)

    out = pl.pallas_call(
        _chain_topk_kernel,
        grid=(nb,),
        in_specs=[
            pl.BlockSpec((length - 1, 1, n, n), lambda b: (0, b, 0, 0)),
            pl.BlockSpec((1, 1, n), lambda b: (b, 0, 0)),
        ],
        out_specs=pl.BlockSpec((1, 1, 2 * _TOPN), lambda b: (b, 0, 0)),
        out_shape=jax.ShapeDtypeStruct((nb, 1, 2 * _TOPN), jnp.int32),
        compiler_params=pltpu.CompilerParams(
            dimension_semantics=("arbitrary",)),
    )(xr, v0)

    out = out.reshape(bsz, heads, 2 * _TOPN)
    shift = jnp.asarray(topn, jnp.int32) - _TOPN
    out0 = out[:, :, :_TOPN].reshape(bsz, heads * _TOPN)
    out1 = out[:, :, _TOPN:].reshape(bsz, heads * _TOPN)
    return jnp.concatenate([out0 + shift, out1 + shift], axis=1)


# speedup vs baseline: 1.6346x; 1.2044x over previous
import jax
import jax.numpy as jnp
from jax.experimental import pallas as pl
from jax.experimental.pallas import tpu as pltpu

_TOPN = 12
_G = 2


def _chain_topk_kernel(x_ref, v0_ref, out_ref):
    accs = [x_ref[0, g] for g in range(_G)]
    for t in range(1, 11):
        for g in range(_G):
            a = x_ref[t, g].astype(jnp.bfloat16)
            accs[g] = jax.lax.dot_general(
                a, accs[g].astype(jnp.bfloat16), (((1,), (0,)), ((), ())),
                preferred_element_type=jnp.float32)

    rows = []
    for g in range(_G):
        v = v0_ref[0, g:g + 1, :].astype(jnp.bfloat16)
        rows.append(jax.lax.dot_general(
            v, accs[g].astype(jnp.bfloat16), (((1,), (0,)), ((), ())),
            preferred_element_type=jnp.float32))
    row = jnp.concatenate(rows, axis=0)

    def topk_indices(seg, base):
        idxs = jax.lax.broadcasted_iota(jnp.int32, seg.shape, 1)
        picks = []
        cur = seg
        for _ in range(_TOPN):
            mx = jnp.max(cur, axis=1, keepdims=True)
            ind = jnp.min(
                jnp.where(cur == mx, idxs, jnp.int32(2**30)),
                axis=1, keepdims=True)
            picks.append(ind + base)
            cur = jnp.where(idxs == ind, -jnp.inf, cur)
        return picks

    p0 = topk_indices(row[:, 1:197], 1)
    p1 = topk_indices(row[:, 198:394], 198)
    out_ref[0] = jnp.concatenate(p0 + p1, axis=1).astype(jnp.int32)


def kernel(x, topn):
    length, bsz, heads, n, _ = x.shape
    nb = bsz * heads
    xr = x.reshape(length, nb, n, n)
    v0 = xr[length - 1, :, 0, :].reshape(nb // _G, _G, n)

    out = pl.pallas_call(
        _chain_topk_kernel,
        grid=(nb // _G,),
        in_specs=[
            pl.BlockSpec((length - 1, _G, n, n), lambda b: (0, b, 0, 0)),
            pl.BlockSpec((1, _G, n), lambda b: (b, 0, 0)),
        ],
        out_specs=pl.BlockSpec((1, _G, 2 * _TOPN), lambda b: (b, 0, 0)),
        out_shape=jax.ShapeDtypeStruct((nb // _G, _G, 2 * _TOPN), jnp.int32),
        compiler_params=pltpu.CompilerParams(
            dimension_semantics=("arbitrary",)),
    )(xr, v0)

    out = out.reshape(bsz, heads, 2 * _TOPN)
    shift = jnp.asarray(topn, jnp.int32) - _TOPN
    out0 = out[:, :, :_TOPN].reshape(bsz, heads * _TOPN)
    out1 = out[:, :, _TOPN:].reshape(bsz, heads * _TOPN)
    return jnp.concatenate([out0 + shift, out1 + shift], axis=1)
